# Initial kernel scaffold; baseline (speedup 1.0000x reference)
import jax
import jax.numpy as jnp
from jax.experimental import pallas as pl


def kernel(inputs, indexes, features, labels):
    raise NotImplementedError("write your pallas kernel here")


# speedup vs baseline: 4.8598x; 4.8598x over previous
import functools

import jax
import jax.numpy as jnp
from jax import lax
from jax.experimental import pallas as pl
from jax.experimental.pallas import tpu as pltpu
from jax.experimental.pallas import tpu_sc as plsc

NUM_FEATURES = 64
NUM_SAMPLES = 100000
NUM_CLASSES = 751
TEMP = 0.05
B = 1024

NC = 2
NS = 16
NW = NC * NS

CHUNK = 128
FULL_CHUNKS = NUM_SAMPLES // CHUNK
TAIL = NUM_SAMPLES - FULL_CHUNKS * CHUNK
KMAIN = FULL_CHUNKS // NW
EXTRA = FULL_CHUNKS - KMAIN * NW
ROWS_PAD = 752
GW = ROWS_PAD * NUM_FEATURES
CPAD = 768
GSTRIPE = GW // NS
CSTRIPE = CPAD // NS
IDX_PER_W = B // NW


def _iota16():
    return lax.iota(jnp.int32, 16)


def _sc_body(feat_hbm, lab_hbm, idx_hbm, g_out, c_out, t_out,
             featv, labv, g_flat, c_flat, tmpv, accv, ctmp, cacc,
             idxraw, idxv, tgtv, stage_g, stage_c, sem):
    cid = lax.axis_index("c")
    sid = lax.axis_index("s")
    w = sid * NC + cid
    it = _iota16()

    zf = jnp.zeros((16,), jnp.float32)

    def zbody(k, carry):
        base = k * 64
        for q in range(4):
            plsc.store_scatter(g_flat, [base + q * 16 + it], zf)
        return carry
    lax.fori_loop(0, GW // 64, zbody, 0)
    for m in range(CPAD // 16):
        c_flat[pl.ds(m * 16, 16)] = zf

    one16 = jnp.ones((16,), jnp.float32)
    lane0 = it == 0

    def accumulate(nsamples):
        def sbody(i, carry):
            lblv = plsc.load_gather(labv, [jnp.full((16,), i, jnp.int32)])
            base = lblv * NUM_FEATURES
            src0 = i * NUM_FEATURES + it
            for j in range(NUM_FEATURES // 16):
                row = plsc.load_gather(featv, [src0 + j * 16])
                plsc.addupdate_scatter(g_flat, [base + j * 16 + it], row)
            plsc.addupdate_scatter(c_flat, [lblv], one16, mask=lane0)
            return carry
        lax.fori_loop(0, nsamples, sbody, 0)

    def do_chunk(srow, nsamples):
        srow = pl.multiple_of(srow, CHUNK)
        pltpu.sync_copy(feat_hbm.at[pl.ds(srow * NUM_FEATURES, nsamples * NUM_FEATURES)],
                        featv if nsamples == CHUNK else featv.at[pl.ds(0, nsamples * NUM_FEATURES)])
        pltpu.sync_copy(lab_hbm.at[pl.ds(srow, nsamples)],
                        labv if nsamples == CHUNK else labv.at[pl.ds(0, nsamples)])
        accumulate(nsamples)

    def cbody(k, carry):
        do_chunk((w + NW * k) * CHUNK, CHUNK)
        return carry
    lax.fori_loop(0, KMAIN, cbody, 0)

    @pl.when(w < EXTRA)
    def _():
        do_chunk((KMAIN * NW + w) * CHUNK, CHUNK)

    @pl.when(w == EXTRA)
    def _():
        do_chunk(FULL_CHUNKS * CHUNK, TAIL)

    pltpu.sync_copy(g_flat, stage_g.at[pl.ds(pl.multiple_of(sid * GW, 128), GW)])
    pltpu.sync_copy(c_flat, stage_c.at[pl.ds(pl.multiple_of(sid * CPAD, 128), CPAD)])

    ibase = pl.multiple_of(w * IDX_PER_W, IDX_PER_W)
    pltpu.sync_copy(idx_hbm.at[pl.ds(ibase, IDX_PER_W)], idxraw)
    for j in range(IDX_PER_W // 16):
        v = idxraw[pl.ds(j * 16, 16)] - 1
        v = jnp.where(v >= 0, v, 0)
        v = jnp.where(v == 5554, NUM_CLASSES - 1, v)
        idxv[pl.ds(j * 16, 16)] = v
    pltpu.async_copy(lab_hbm.at[idxv], tgtv, sem).wait()
    pltpu.sync_copy(tgtv, t_out.at[pl.ds(ibase, IDX_PER_W)])

    plsc.subcore_barrier()

    gs = sid * GSTRIPE
    pltpu.sync_copy(stage_g.at[pl.ds(pl.multiple_of(gs, 8), GSTRIPE)], accv)
    for b in range(1, NS):
        pltpu.sync_copy(stage_g.at[pl.ds(pl.multiple_of(b * GW + gs, 8), GSTRIPE)], tmpv)

        def rbody(m, carry):
            idx = m * 16 + it
            acc = plsc.load_gather(accv, [idx]) + plsc.load_gather(tmpv, [idx])
            plsc.store_scatter(accv, [idx], acc)
            return carry
        lax.fori_loop(0, GSTRIPE // 16, rbody, 0)

    cs = sid * CSTRIPE
    pltpu.sync_copy(stage_c.at[pl.ds(pl.multiple_of(cs, 8), CSTRIPE)], cacc)
    for b in range(1, NS):
        pltpu.sync_copy(stage_c.at[pl.ds(pl.multiple_of(b * CPAD + cs, 8), CSTRIPE)], ctmp)
        for m in range(CSTRIPE // 16):
            idx = m * 16 + it
            acc = plsc.load_gather(cacc, [idx]) + plsc.load_gather(ctmp, [idx])
            cacc[pl.ds(m * 16, 16)] = acc

    pltpu.sync_copy(accv, g_out.at[pl.ds(pl.multiple_of(cid * GW + gs, 8), GSTRIPE)])
    pltpu.sync_copy(cacc, c_out.at[pl.ds(pl.multiple_of(cid * CPAD + cs, 8), CSTRIPE)])


@functools.cache
def _make_sc_seg():
    return pl.kernel(
        _sc_body,
        out_type=(
            jax.ShapeDtypeStruct((NC * GW,), jnp.float32),
            jax.ShapeDtypeStruct((NC * CPAD,), jnp.float32),
            jax.ShapeDtypeStruct((B,), jnp.int32),
        ),
        compiler_params=pltpu.CompilerParams(needs_layout_passes=False),
        mesh=plsc.VectorSubcoreMesh(core_axis_name="c", subcore_axis_name="s",
                                    num_cores=NC, num_subcores=NS),
        scratch_types=[
            pltpu.VMEM((CHUNK * NUM_FEATURES,), jnp.float32),
            pltpu.VMEM((CHUNK,), jnp.int32),
            pltpu.VMEM((GW,), jnp.float32),
            pltpu.VMEM((CPAD,), jnp.float32),
            pltpu.VMEM((GSTRIPE,), jnp.float32),
            pltpu.VMEM((GSTRIPE,), jnp.float32),
            pltpu.VMEM((CSTRIPE,), jnp.float32),
            pltpu.VMEM((CSTRIPE,), jnp.float32),
            pltpu.VMEM((IDX_PER_W,), jnp.int32),
            pltpu.VMEM((IDX_PER_W,), jnp.int32),
            pltpu.VMEM((IDX_PER_W,), jnp.int32),
            pltpu.VMEM_SHARED((NS * GW,), jnp.float32),
            pltpu.VMEM_SHARED((NS * CPAD,), jnp.float32),
            pltpu.SemaphoreType.DMA,
        ],
    )


def _tc_body(x_ref, idx_ref, tgt_ref, g_ref, c_ref, o_ref):
    x = x_ref[...]
    nrm = jnp.sqrt(jnp.sum(x * x, axis=1, keepdims=True))
    x = x / jnp.clip(nrm, 1e-12, None)
    g = (g_ref[0] + g_ref[1])[:NUM_CLASSES]
    cnt_row = (c_ref[0] + c_ref[1])[:NUM_CLASSES].reshape(1, NUM_CLASSES)
    sim = lax.dot_general(x, g, (((1,), (1,)), ((), ())),
                          preferred_element_type=jnp.float32)
    pos = cnt_row > 0
    scale = jnp.where(pos, 1.0 / (TEMP * cnt_row), 0.0)
    exps = jnp.exp(sim * scale) * pos.astype(jnp.float32)
    sums = jnp.sum(exps, axis=1, keepdims=True) + 1e-6
    logp = jnp.log(exps / sums + 1e-6)
    tgt = tgt_ref[...]
    cols = lax.broadcasted_iota(jnp.int32, (B, NUM_CLASSES), 1)
    picked = jnp.sum(jnp.where(cols == tgt, logp, 0.0), axis=1, keepdims=True)
    keep = (idx_ref[...] - 1) >= 0
    valid = jnp.logical_and(tgt != NUM_CLASSES - 1, keep).astype(jnp.float32)
    num = jnp.sum(picked * valid)
    den = jnp.maximum(jnp.sum(valid), 1.0)
    o_ref[...] = jnp.broadcast_to(-num / den, (1, 1))


def kernel(inputs, indexes, features, labels):
    idx32 = indexes.astype(jnp.int32)
    g_parts, c_parts, targets = _make_sc_seg()(
        features.reshape(-1), labels.astype(jnp.int32), idx32)
    g_parts = g_parts.reshape(NC, ROWS_PAD, NUM_FEATURES)
    c_parts = c_parts.reshape(NC, CPAD)
    loss = pl.pallas_call(
        _tc_body,
        out_shape=jax.ShapeDtypeStruct((1, 1), jnp.float32),
    )(inputs, idx32.reshape(B, 1), targets.reshape(B, 1), g_parts, c_parts)
    return loss[0, 0]


# speedup vs baseline: 6.0909x; 1.2533x over previous
import functools

import jax
import jax.numpy as jnp
from jax import lax
from jax.experimental import pallas as pl
from jax.experimental.pallas import tpu as pltpu
from jax.experimental.pallas import tpu_sc as plsc

NUM_FEATURES = 64
NUM_SAMPLES = 100000
NUM_CLASSES = 751
TEMP = 0.05
B = 1024

NC = 2
NS = 16
NW = NC * NS

CHUNK = 128
FULL_CHUNKS = NUM_SAMPLES // CHUNK
TAIL = NUM_SAMPLES - FULL_CHUNKS * CHUNK
KMAIN = FULL_CHUNKS // NW
EXTRA = FULL_CHUNKS - KMAIN * NW
ROWS_PAD = 752
GW = ROWS_PAD * NUM_FEATURES
CPAD = 768
GSTRIPE = GW // NS
CSTRIPE = CPAD // NS
IDX_PER_W = B // NW


def _iota16():
    return lax.iota(jnp.int32, 16)


FB0 = 0
FB1 = CHUNK * NUM_FEATURES
ACC = 0
TMP0 = GSTRIPE
TMP1 = 2 * GSTRIPE
CAC = 3 * GSTRIPE
CTM = 3 * GSTRIPE + CSTRIPE
BUFW = 2 * CHUNK * NUM_FEATURES


def _sc_body(feat_hbm, lab_hbm, idx_hbm, g_out, c_out, t_out,
             buf, labv0, labv1, g_flat, c_flat,
             idxraw, idxv, tgtv, stage_g, stage_c,
             semf0, semf1, seml0, seml1, semt0, semt1, semx):
    cid = lax.axis_index("c")
    sid = lax.axis_index("s")
    w = sid * NC + cid
    it = _iota16()
    fbase = (FB0, FB1)
    labb = (labv0, labv1)
    semfb = (semf0, semf1)
    semlb = (seml0, seml1)

    def chunk_row(k):
        return pl.multiple_of((w + NW * k) * CHUNK, CHUNK)

    def fdst(b, n=CHUNK):
        return buf.at[pl.ds(fbase[b], n * NUM_FEATURES)]

    def fsrc(k, n=CHUNK):
        return feat_hbm.at[pl.ds(chunk_row(k) * NUM_FEATURES, n * NUM_FEATURES)]

    def lsrc(k, n=CHUNK):
        return lab_hbm.at[pl.ds(chunk_row(k), n)]

    for b in range(2):
        pltpu.async_copy(fsrc(b), fdst(b), semfb[b])
        pltpu.async_copy(lsrc(b), labb[b], semlb[b])

    zf = jnp.zeros((16,), jnp.float32)

    def zbody(k, carry):
        base = k * 64
        for q in range(4):
            plsc.store_scatter(g_flat, [base + q * 16 + it], zf)
        return carry
    lax.fori_loop(0, GW // 64, zbody, 0)
    for m in range(CPAD // 16):
        c_flat[pl.ds(m * 16, 16)] = zf

    one16 = jnp.ones((16,), jnp.float32)
    lane0 = it == 0
    cols = [j * 16 + it for j in range(NUM_FEATURES // 16)]

    def sample(b, lv, i):
        src0 = fbase[b] + i * NUM_FEATURES
        lblv = plsc.load_gather(lv, [jnp.full((16,), i, jnp.int32)])
        base = lblv * NUM_FEATURES
        for j in range(NUM_FEATURES // 16):
            row = plsc.load_gather(buf, [src0 + cols[j]])
            plsc.addupdate_scatter(g_flat, [base + cols[j]], row)
        plsc.addupdate_scatter(c_flat, [lblv], one16, mask=lane0)

    def accumulate(b, nsamples):
        lv = labb[b]

        def sbody(q, carry):
            for u in range(4):
                sample(b, lv, q * 4 + u)
            return carry
        lax.fori_loop(0, nsamples // 4, sbody, 0)

    def cbody(k2, carry):
        for b in range(2):
            k = k2 * 2 + b
            pltpu.make_async_copy(fsrc(k), fdst(b), semfb[b]).wait()
            pltpu.make_async_copy(lsrc(k), labb[b], semlb[b]).wait()
            accumulate(b, CHUNK)

            @pl.when(k + 2 < KMAIN)
            def _():
                pltpu.async_copy(fsrc(k + 2), fdst(b), semfb[b])
                pltpu.async_copy(lsrc(k + 2), labb[b], semlb[b])
        return carry
    lax.fori_loop(0, KMAIN // 2, cbody, 0)

    @pl.when(w < EXTRA)
    def _():
        r = pl.multiple_of((KMAIN * NW + w) * CHUNK * NUM_FEATURES, CHUNK)
        s = feat_hbm.at[pl.ds(r, CHUNK * NUM_FEATURES)]
        sl = lab_hbm.at[pl.ds(pl.multiple_of((KMAIN * NW + w) * CHUNK, CHUNK), CHUNK)]
        pltpu.async_copy(s, fdst(0), semf0)
        pltpu.async_copy(sl, labv0, seml0)
        pltpu.make_async_copy(s, fdst(0), semf0).wait()
        pltpu.make_async_copy(sl, labv0, seml0).wait()
        accumulate(0, CHUNK)

    @pl.when(w == EXTRA)
    def _():
        s = feat_hbm.at[pl.ds(FULL_CHUNKS * CHUNK * NUM_FEATURES, TAIL * NUM_FEATURES)]
        sl = lab_hbm.at[pl.ds(FULL_CHUNKS * CHUNK, TAIL)]
        pltpu.async_copy(s, fdst(0, TAIL), semf0)
        pltpu.async_copy(sl, labv0.at[pl.ds(0, TAIL)], seml0)
        pltpu.make_async_copy(s, fdst(0, TAIL), semf0).wait()
        pltpu.make_async_copy(sl, labv0.at[pl.ds(0, TAIL)], seml0).wait()
        accumulate(0, TAIL)

    pltpu.sync_copy(g_flat, stage_g.at[pl.ds(pl.multiple_of(sid * GW, 128), GW)])
    pltpu.sync_copy(c_flat, stage_c.at[pl.ds(pl.multiple_of(sid * CPAD, 128), CPAD)])

    ibase = pl.multiple_of(w * IDX_PER_W, IDX_PER_W)
    pltpu.sync_copy(idx_hbm.at[pl.ds(ibase, IDX_PER_W)], idxraw)
    for j in range(IDX_PER_W // 16):
        v = idxraw[pl.ds(j * 16, 16)] - 1
        v = jnp.where(v >= 0, v, 0)
        v = jnp.where(v == 5554, NUM_CLASSES - 1, v)
        idxv[pl.ds(j * 16, 16)] = v
    pltpu.async_copy(lab_hbm.at[idxv], tgtv, semx).wait()
    pltpu.sync_copy(tgtv, t_out.at[pl.ds(ibase, IDX_PER_W)])

    plsc.subcore_barrier()

    gs = sid * GSTRIPE
    tmpb = (TMP0, TMP1)
    semtb = (semt0, semt1)

    def gsl(b):
        return stage_g.at[pl.ds(pl.multiple_of(b * GW + gs, 8), GSTRIPE)]

    def bsl(off):
        return buf.at[pl.ds(off, GSTRIPE)]
    pltpu.async_copy(gsl(0), bsl(ACC), semx)
    pltpu.async_copy(gsl(1), bsl(TMP0), semt0)
    pltpu.make_async_copy(gsl(0), bsl(ACC), semx).wait()
    for b in range(1, NS):
        p = (b - 1) % 2
        pltpu.make_async_copy(gsl(b), bsl(tmpb[p]), semtb[p]).wait()
        if b + 1 < NS:
            pltpu.async_copy(gsl(b + 1), bsl(tmpb[b % 2]), semtb[b % 2])

        def rbody(m, carry):
            idx = m * 16 + it
            acc = plsc.load_gather(buf, [ACC + idx]) + plsc.load_gather(buf, [tmpb[p] + idx])
            plsc.store_scatter(buf, [ACC + idx], acc)
            return carry
        lax.fori_loop(0, GSTRIPE // 16, rbody, 0)

    cs = sid * CSTRIPE

    def csl(b):
        return stage_c.at[pl.ds(pl.multiple_of(b * CPAD + cs, 8), CSTRIPE)]
    pltpu.sync_copy(csl(0), buf.at[pl.ds(CAC, CSTRIPE)])
    for b in range(1, NS):
        pltpu.sync_copy(csl(b), buf.at[pl.ds(CTM, CSTRIPE)])
        for m in range(CSTRIPE // 16):
            idx = m * 16 + it
            acc = plsc.load_gather(buf, [CAC + idx]) + plsc.load_gather(buf, [CTM + idx])
            plsc.store_scatter(buf, [CAC + idx], acc)

    pltpu.sync_copy(bsl(ACC), g_out.at[pl.ds(pl.multiple_of(cid * GW + gs, 8), GSTRIPE)])
    pltpu.sync_copy(buf.at[pl.ds(CAC, CSTRIPE)],
                    c_out.at[pl.ds(pl.multiple_of(cid * CPAD + cs, 8), CSTRIPE)])


@functools.cache
def _make_sc_seg():
    return pl.kernel(
        _sc_body,
        out_type=(
            jax.ShapeDtypeStruct((NC * GW,), jnp.float32),
            jax.ShapeDtypeStruct((NC * CPAD,), jnp.float32),
            jax.ShapeDtypeStruct((B,), jnp.int32),
        ),
        compiler_params=pltpu.CompilerParams(needs_layout_passes=False),
        mesh=plsc.VectorSubcoreMesh(core_axis_name="c", subcore_axis_name="s",
                                    num_cores=NC, num_subcores=NS),
        scratch_types=[
            pltpu.VMEM((BUFW,), jnp.float32),
            pltpu.VMEM((CHUNK,), jnp.int32),
            pltpu.VMEM((CHUNK,), jnp.int32),
            pltpu.VMEM((GW,), jnp.float32),
            pltpu.VMEM((CPAD,), jnp.float32),
            pltpu.VMEM((IDX_PER_W,), jnp.int32),
            pltpu.VMEM((IDX_PER_W,), jnp.int32),
            pltpu.VMEM((IDX_PER_W,), jnp.int32),
            pltpu.VMEM_SHARED((NS * GW,), jnp.float32),
            pltpu.VMEM_SHARED((NS * CPAD,), jnp.float32),
            pltpu.SemaphoreType.DMA,
            pltpu.SemaphoreType.DMA,
            pltpu.SemaphoreType.DMA,
            pltpu.SemaphoreType.DMA,
            pltpu.SemaphoreType.DMA,
            pltpu.SemaphoreType.DMA,
            pltpu.SemaphoreType.DMA,
        ],
    )


def _tc_body(x_ref, idx_ref, tgt_ref, g_ref, c_ref, o_ref):
    x = x_ref[...]
    nrm = jnp.sqrt(jnp.sum(x * x, axis=1, keepdims=True))
    x = x / jnp.clip(nrm, 1e-12, None)
    g = (g_ref[0] + g_ref[1])[:NUM_CLASSES]
    cnt_row = (c_ref[0] + c_ref[1])[:NUM_CLASSES].reshape(1, NUM_CLASSES)
    sim = lax.dot_general(x, g, (((1,), (1,)), ((), ())),
                          preferred_element_type=jnp.float32)
    pos = cnt_row > 0
    scale = jnp.where(pos, 1.0 / (TEMP * cnt_row), 0.0)
    exps = jnp.exp(sim * scale) * pos.astype(jnp.float32)
    sums = jnp.sum(exps, axis=1, keepdims=True) + 1e-6
    logp = jnp.log(exps / sums + 1e-6)
    tgt = tgt_ref[...]
    cols = lax.broadcasted_iota(jnp.int32, (B, NUM_CLASSES), 1)
    picked = jnp.sum(jnp.where(cols == tgt, logp, 0.0), axis=1, keepdims=True)
    keep = (idx_ref[...] - 1) >= 0
    valid = jnp.logical_and(tgt != NUM_CLASSES - 1, keep).astype(jnp.float32)
    num = jnp.sum(picked * valid)
    den = jnp.maximum(jnp.sum(valid), 1.0)
    o_ref[...] = jnp.broadcast_to(-num / den, (1, 1))


def kernel(inputs, indexes, features, labels):
    idx32 = indexes.astype(jnp.int32)
    g_parts, c_parts, targets = _make_sc_seg()(
        features.reshape(-1), labels.astype(jnp.int32), idx32)
    g_parts = g_parts.reshape(NC, ROWS_PAD, NUM_FEATURES)
    c_parts = c_parts.reshape(NC, CPAD)
    loss = pl.pallas_call(
        _tc_body,
        out_shape=jax.ShapeDtypeStruct((1, 1), jnp.float32),
    )(inputs, idx32.reshape(B, 1), targets.reshape(B, 1), g_parts, c_parts)
    return loss[0, 0]


# speedup vs baseline: 8.4959x; 1.3949x over previous
import functools

import jax
import jax.numpy as jnp
from jax import lax
from jax.experimental import pallas as pl
from jax.experimental.pallas import tpu as pltpu
from jax.experimental.pallas import tpu_sc as plsc

NUM_FEATURES = 64
NUM_SAMPLES = 100000
NUM_CLASSES = 751
TEMP = 0.05
B = 1024

NC = 2
NS = 16
NW = NC * NS

CHUNK = 128
FULL_CHUNKS = NUM_SAMPLES // CHUNK
TAIL = NUM_SAMPLES - FULL_CHUNKS * CHUNK
KMAIN = FULL_CHUNKS // NW
EXTRA = FULL_CHUNKS - KMAIN * NW
ROWS_PAD = 752
GW = ROWS_PAD * NUM_FEATURES
CPAD = 768
GSTRIPE = GW // NS
CSTRIPE = CPAD // NS
IDX_PER_W = B // NW


def _iota16():
    return lax.iota(jnp.int32, 16)


FB0 = 0
FB1 = CHUNK * NUM_FEATURES
ACC = 0
TMP0 = GSTRIPE
TMP1 = 2 * GSTRIPE
CAC = 3 * GSTRIPE
CTM = 3 * GSTRIPE + CSTRIPE
BUFW = 2 * CHUNK * NUM_FEATURES


def _sc_body(feat_hbm, lab_hbm, idx_hbm, g_out, c_out, t_out,
             buf, labv0, labv1, g_flat, c_flat,
             idxraw, idxv, tgtv, stage_g, stage_c,
             semf0, semf1, seml0, seml1, semt0, semt1, semx):
    cid = lax.axis_index("c")
    sid = lax.axis_index("s")
    w = sid * NC + cid
    it = _iota16()
    fbase = (FB0, FB1)
    labb = (labv0, labv1)
    semfb = (semf0, semf1)
    semlb = (seml0, seml1)

    def chunk_row(k):
        return pl.multiple_of((w + NW * k) * CHUNK, CHUNK)

    def fdst(b, n=CHUNK):
        return buf.at[pl.ds(fbase[b], n * NUM_FEATURES)]

    def fsrc(k, n=CHUNK):
        return feat_hbm.at[pl.ds(chunk_row(k) * NUM_FEATURES, n * NUM_FEATURES)]

    def lsrc(k, n=CHUNK):
        return lab_hbm.at[pl.ds(chunk_row(k), n)]

    for b in range(2):
        pltpu.async_copy(fsrc(b), fdst(b), semfb[b])
        pltpu.async_copy(lsrc(b), labb[b], semlb[b])

    zf = jnp.zeros((16,), jnp.float32)

    @plsc.parallel_loop(0, GW // 64, unroll=2)
    def _(k):
        base = k * 64
        for q in range(4):
            plsc.store_scatter(g_flat, [base + q * 16 + it], zf)
    for m in range(CPAD // 16):
        c_flat[pl.ds(m * 16, 16)] = zf

    one16 = jnp.ones((16,), jnp.float32)
    lane0 = it == 0
    cols = [j * 16 + it for j in range(NUM_FEATURES // 16)]

    def sample(b, lv, i):
        src0 = fbase[b] + i * NUM_FEATURES
        lblv = plsc.load_gather(lv, [jnp.full((16,), i, jnp.int32)])
        base = lblv * NUM_FEATURES
        for j in range(NUM_FEATURES // 16):
            row = plsc.load_gather(buf, [src0 + cols[j]])
            plsc.addupdate_scatter(g_flat, [base + cols[j]], row)
        plsc.addupdate_scatter(c_flat, [lblv], one16, mask=lane0)

    def accumulate(b, nsamples):
        lv = labb[b]

        @plsc.parallel_loop(0, nsamples // 4, unroll=2)
        def _(q):
            for u in range(4):
                sample(b, lv, q * 4 + u)

    def cbody(k2, carry):
        for b in range(2):
            k = k2 * 2 + b
            pltpu.make_async_copy(fsrc(k), fdst(b), semfb[b]).wait()
            pltpu.make_async_copy(lsrc(k), labb[b], semlb[b]).wait()
            accumulate(b, CHUNK)

            @pl.when(k + 2 < KMAIN)
            def _():
                pltpu.async_copy(fsrc(k + 2), fdst(b), semfb[b])
                pltpu.async_copy(lsrc(k + 2), labb[b], semlb[b])
        return carry
    lax.fori_loop(0, KMAIN // 2, cbody, 0)

    @pl.when(w < EXTRA)
    def _():
        r = pl.multiple_of((KMAIN * NW + w) * CHUNK * NUM_FEATURES, CHUNK)
        s = feat_hbm.at[pl.ds(r, CHUNK * NUM_FEATURES)]
        sl = lab_hbm.at[pl.ds(pl.multiple_of((KMAIN * NW + w) * CHUNK, CHUNK), CHUNK)]
        pltpu.async_copy(s, fdst(0), semf0)
        pltpu.async_copy(sl, labv0, seml0)
        pltpu.make_async_copy(s, fdst(0), semf0).wait()
        pltpu.make_async_copy(sl, labv0, seml0).wait()
        accumulate(0, CHUNK)

    @pl.when(w == EXTRA)
    def _():
        s = feat_hbm.at[pl.ds(FULL_CHUNKS * CHUNK * NUM_FEATURES, TAIL * NUM_FEATURES)]
        sl = lab_hbm.at[pl.ds(FULL_CHUNKS * CHUNK, TAIL)]
        pltpu.async_copy(s, fdst(0, TAIL), semf0)
        pltpu.async_copy(sl, labv0.at[pl.ds(0, TAIL)], seml0)
        pltpu.make_async_copy(s, fdst(0, TAIL), semf0).wait()
        pltpu.make_async_copy(sl, labv0.at[pl.ds(0, TAIL)], seml0).wait()
        accumulate(0, TAIL)

    pltpu.sync_copy(g_flat, stage_g.at[pl.ds(pl.multiple_of(sid * GW, 128), GW)])
    pltpu.sync_copy(c_flat, stage_c.at[pl.ds(pl.multiple_of(sid * CPAD, 128), CPAD)])

    ibase = pl.multiple_of(w * IDX_PER_W, IDX_PER_W)
    pltpu.sync_copy(idx_hbm.at[pl.ds(ibase, IDX_PER_W)], idxraw)
    for j in range(IDX_PER_W // 16):
        v = idxraw[pl.ds(j * 16, 16)] - 1
        v = jnp.where(v >= 0, v, 0)
        v = jnp.where(v == 5554, NUM_CLASSES - 1, v)
        idxv[pl.ds(j * 16, 16)] = v
    pltpu.async_copy(lab_hbm.at[idxv], tgtv, semx).wait()
    pltpu.sync_copy(tgtv, t_out.at[pl.ds(ibase, IDX_PER_W)])

    plsc.subcore_barrier()

    gs = sid * GSTRIPE
    tmpb = (TMP0, TMP1)
    semtb = (semt0, semt1)

    def gsl(b):
        return stage_g.at[pl.ds(pl.multiple_of(b * GW + gs, 8), GSTRIPE)]

    def bsl(off):
        return buf.at[pl.ds(off, GSTRIPE)]
    pltpu.async_copy(gsl(0), bsl(ACC), semx)
    pltpu.async_copy(gsl(1), bsl(TMP0), semt0)
    pltpu.make_async_copy(gsl(0), bsl(ACC), semx).wait()
    for b in range(1, NS):
        p = (b - 1) % 2
        pltpu.make_async_copy(gsl(b), bsl(tmpb[p]), semtb[p]).wait()
        if b + 1 < NS:
            pltpu.async_copy(gsl(b + 1), bsl(tmpb[b % 2]), semtb[b % 2])

        @plsc.parallel_loop(0, GSTRIPE // 16, unroll=4)
        def _(m):
            idx = m * 16 + it
            acc = plsc.load_gather(buf, [ACC + idx]) + plsc.load_gather(buf, [tmpb[p] + idx])
            plsc.store_scatter(buf, [ACC + idx], acc)

    cs = sid * CSTRIPE

    def csl(b):
        return stage_c.at[pl.ds(pl.multiple_of(b * CPAD + cs, 8), CSTRIPE)]
    pltpu.sync_copy(csl(0), buf.at[pl.ds(CAC, CSTRIPE)])
    for b in range(1, NS):
        pltpu.sync_copy(csl(b), buf.at[pl.ds(CTM, CSTRIPE)])
        for m in range(CSTRIPE // 16):
            idx = m * 16 + it
            acc = plsc.load_gather(buf, [CAC + idx]) + plsc.load_gather(buf, [CTM + idx])
            plsc.store_scatter(buf, [CAC + idx], acc)

    pltpu.sync_copy(bsl(ACC), g_out.at[pl.ds(pl.multiple_of(cid * GW + gs, 8), GSTRIPE)])
    pltpu.sync_copy(buf.at[pl.ds(CAC, CSTRIPE)],
                    c_out.at[pl.ds(pl.multiple_of(cid * CPAD + cs, 8), CSTRIPE)])


@functools.cache
def _make_sc_seg():
    return pl.kernel(
        _sc_body,
        out_type=(
            jax.ShapeDtypeStruct((NC * GW,), jnp.float32),
            jax.ShapeDtypeStruct((NC * CPAD,), jnp.float32),
            jax.ShapeDtypeStruct((B,), jnp.int32),
        ),
        compiler_params=pltpu.CompilerParams(needs_layout_passes=False),
        mesh=plsc.VectorSubcoreMesh(core_axis_name="c", subcore_axis_name="s",
                                    num_cores=NC, num_subcores=NS),
        scratch_types=[
            pltpu.VMEM((BUFW,), jnp.float32),
            pltpu.VMEM((CHUNK,), jnp.int32),
            pltpu.VMEM((CHUNK,), jnp.int32),
            pltpu.VMEM((GW,), jnp.float32),
            pltpu.VMEM((CPAD,), jnp.float32),
            pltpu.VMEM((IDX_PER_W,), jnp.int32),
            pltpu.VMEM((IDX_PER_W,), jnp.int32),
            pltpu.VMEM((IDX_PER_W,), jnp.int32),
            pltpu.VMEM_SHARED((NS * GW,), jnp.float32),
            pltpu.VMEM_SHARED((NS * CPAD,), jnp.float32),
            pltpu.SemaphoreType.DMA,
            pltpu.SemaphoreType.DMA,
            pltpu.SemaphoreType.DMA,
            pltpu.SemaphoreType.DMA,
            pltpu.SemaphoreType.DMA,
            pltpu.SemaphoreType.DMA,
            pltpu.SemaphoreType.DMA,
        ],
    )


def _tc_body(x_ref, idx_ref, tgt_ref, g_ref, c_ref, o_ref):
    x = x_ref[...]
    nrm = jnp.sqrt(jnp.sum(x * x, axis=1, keepdims=True))
    x = x / jnp.clip(nrm, 1e-12, None)
    g = (g_ref[0] + g_ref[1])[:NUM_CLASSES]
    cnt_row = (c_ref[0] + c_ref[1])[:NUM_CLASSES].reshape(1, NUM_CLASSES)
    sim = lax.dot_general(x, g, (((1,), (1,)), ((), ())),
                          preferred_element_type=jnp.float32)
    pos = cnt_row > 0
    scale = jnp.where(pos, 1.0 / (TEMP * cnt_row), 0.0)
    exps = jnp.exp(sim * scale) * pos.astype(jnp.float32)
    sums = jnp.sum(exps, axis=1, keepdims=True) + 1e-6
    logp = jnp.log(exps / sums + 1e-6)
    tgt = tgt_ref[...]
    cols = lax.broadcasted_iota(jnp.int32, (B, NUM_CLASSES), 1)
    picked = jnp.sum(jnp.where(cols == tgt, logp, 0.0), axis=1, keepdims=True)
    keep = (idx_ref[...] - 1) >= 0
    valid = jnp.logical_and(tgt != NUM_CLASSES - 1, keep).astype(jnp.float32)
    num = jnp.sum(picked * valid)
    den = jnp.maximum(jnp.sum(valid), 1.0)
    o_ref[...] = jnp.broadcast_to(-num / den, (1, 1))


def kernel(inputs, indexes, features, labels):
    idx32 = indexes.astype(jnp.int32)
    g_parts, c_parts, targets = _make_sc_seg()(
        features.reshape(-1), labels.astype(jnp.int32), idx32)
    g_parts = g_parts.reshape(NC, ROWS_PAD, NUM_FEATURES)
    c_parts = c_parts.reshape(NC, CPAD)
    loss = pl.pallas_call(
        _tc_body,
        out_shape=jax.ShapeDtypeStruct((1, 1), jnp.float32),
    )(inputs, idx32.reshape(B, 1), targets.reshape(B, 1), g_parts, c_parts)
    return loss[0, 0]
